# baseline (speedup 1.0000x reference)
import dataclasses
import functools

import jax
import jax.numpy as jnp
from jax import lax
from jax.experimental import pallas as pl
from jax.experimental.pallas import tpu as pltpu
from jax.experimental.pallas import tpu_sc as plsc

NC = 2
NS = 16
NW = NC * NS
LANES = 16

EC = 400


_MESH = plsc.VectorSubcoreMesh(
    core_axis_name="c", subcore_axis_name="s", num_cores=NC, num_subcores=NS
)

_SC_PARAMS = pltpu.CompilerParams(
    needs_layout_passes=False,
    use_tc_tiling_on_sc=False,
    internal_scratch_in_bytes=0,
)


def _bcast_lane(vec, j):
  idx = jnp.full((LANES,), j, dtype=jnp.int32)
  return vec.at[idx].get(mode="promise_in_bounds")


def _bcast_scalar(x):
  return jnp.broadcast_to(x, (LANES,))


def _sc_degree(dstf, ewf, n_pad):
  _, epw = dstf.shape

  @functools.partial(
      pl.kernel,
      out_type=jax.ShapeDtypeStruct((NW, n_pad), jnp.float32),
      mesh=_MESH,
      scratch_types=[
          pltpu.VMEM((epw,), jnp.int32),
          pltpu.VMEM((epw,), jnp.float32),
          pltpu.VMEM((n_pad,), jnp.float32),
          pltpu.SemaphoreType.DMA,
      ],
      compiler_params=_SC_PARAMS,
  )
  def deg_kernel(dst_hbm, ewf_hbm, out_hbm, dst_v, ew_v, deg_v, sem):
    c = lax.axis_index("c")
    s = lax.axis_index("s")
    wid = s * NC + c
    pltpu.sync_copy(dst_hbm.at[wid], dst_v)
    pltpu.sync_copy(ewf_hbm.at[wid], ew_v)

    zero16 = jnp.zeros((LANES,), jnp.float32)

    @pl.loop(0, n_pad // LANES)
    def _(i):
      deg_v[pl.ds(i * LANES, LANES)] = zero16

    @pl.loop(0, epw // LANES)
    def _(i):
      idx = dst_v[pl.ds(i * LANES, LANES)]
      w = ew_v[pl.ds(i * LANES, LANES)]
      plsc.addupdate_scatter(deg_v, [idx], w)

    pltpu.sync_copy(deg_v, out_hbm.at[wid])

  return deg_kernel(dstf, ewf)


@functools.lru_cache(maxsize=None)
def _make_agg_kernel(n, d, kw, c_, n_pad):
  dh = d // 2
  epw = kw * c_
  rpt = n_pad // NS
  assert kw % 2 == 0

  @functools.partial(
      pl.kernel,
      out_type=jax.ShapeDtypeStruct((NC, n_pad, dh), jnp.float32),
      mesh=_MESH,
      scratch_types=[
          pltpu.VMEM((2, c_), jnp.int32),
          pltpu.VMEM((2, c_), jnp.int32),
          pltpu.VMEM((epw,), jnp.float32),
          pltpu.VMEM((c_, dh), jnp.float32),
          pltpu.VMEM((c_, dh), jnp.float32),
          pltpu.VMEM_SHARED((n_pad, dh), jnp.float32),
          pltpu.SemaphoreType.DMA,
          pltpu.SemaphoreType.DMA,
      ],
      compiler_params=_SC_PARAMS,
  )
  def agg_kernel(
      ys2_hbm, src_hbm, dst_hbm, ewf_hbm, z_hbm, out_hbm,
      src_v, dst_v, ew_v, rows_a, rows_b, acc_sh, gsem_a, gsem_b,
  ):
    c = lax.axis_index("c")
    s = lax.axis_index("s")
    lane_iota = lax.iota(jnp.int32, LANES)
    pltpu.sync_copy(ewf_hbm.at[s], ew_v)

    row0 = s * rpt
    pltpu.sync_copy(z_hbm, acc_sh.at[pl.ds(row0, rpt), :])
    plsc.subcore_barrier()

    ys_c = ys2_hbm.at[c]

    def scale(buf, i):
      @plsc.parallel_loop(0, c_ // LANES, step=1, unroll=2)
      def _(g):
        ewv = ew_v[pl.ds(i * c_ + g * LANES, LANES)]
        for j in range(LANES):
          bj = _bcast_lane(ewv, j)
          rowv = _bcast_scalar(g * LANES + j)
          for kk in range(dh // LANES):
            col = lane_iota + (kk * LANES)
            val = plsc.load_gather(buf, [rowv, col])
            plsc.store_scatter(buf, [rowv, col], val * bj)

    @pl.loop(0, kw // 2)
    def _(t):
      i0 = 2 * t
      pltpu.sync_copy(src_hbm.at[s, i0], src_v.at[0])
      pltpu.sync_copy(src_hbm.at[s, i0 + 1], src_v.at[1])
      pltpu.sync_copy(dst_hbm.at[s, i0], dst_v.at[0])
      pltpu.sync_copy(dst_hbm.at[s, i0 + 1], dst_v.at[1])
      ga = pltpu.async_copy(ys_c.at[src_v.at[0]], rows_a, gsem_a)
      gb = pltpu.async_copy(ys_c.at[src_v.at[1]], rows_b, gsem_b)
      ga.wait()
      scale(rows_a, i0)
      pltpu.sync_copy(rows_a, acc_sh.at[dst_v.at[0]], add=True)
      gb.wait()
      scale(rows_b, i0 + 1)
      pltpu.sync_copy(rows_b, acc_sh.at[dst_v.at[1]], add=True)

    plsc.subcore_barrier()
    pltpu.sync_copy(
        acc_sh.at[pl.ds(row0, rpt), :], out_hbm.at[c, pl.ds(row0, rpt), :]
    )

  return agg_kernel


def _sc_aggregate(ys2, src2, dst2, ew2, zeros, n_pad):
  _, n, dh = ys2.shape
  _, kw, c_ = src2.shape
  agg_kernel = _make_agg_kernel(n, 2 * dh, kw, c_, n_pad)
  return agg_kernel(ys2, src2, dst2, ew2.reshape(NS, kw * c_), zeros)


def _tc_first(degp, x, w0):
  n, d = x.shape
  dh = d // 2

  def body(degp_ref, x_ref, w_ref, ys2_ref, dinv_ref):
    deg = jnp.sum(degp_ref[:, pl.ds(0, n)], axis=0)[:, None] + 1.0
    dinv = lax.rsqrt(deg)
    dinv_ref[...] = dinv
    xw = jnp.dot(x_ref[...], w_ref[...], preferred_element_type=jnp.float32)
    ys = xw * dinv
    ys2_ref[0] = ys[:, :dh]
    ys2_ref[1] = ys[:, dh:]

  return pl.pallas_call(
      body,
      out_shape=(
          jax.ShapeDtypeStruct((2, n, dh), jnp.float32),
          jax.ShapeDtypeStruct((n, 1), jnp.float32),
      ),
  )(degp, x, w0)


def _tc_mid(accp, ys2, dinv, b, w_next):
  _, n, dh = ys2.shape

  def body(acc_ref, ys2_ref, dinv_ref, b_ref, w_ref, o2_ref):
    dinv = dinv_ref[...]
    aggl = acc_ref[0, pl.ds(0, n), :] + ys2_ref[0]
    aggr = acc_ref[1, pl.ds(0, n), :] + ys2_ref[1]
    tot = jnp.concatenate([aggl, aggr], axis=1) * dinv + b_ref[...][None, :]
    h = jnp.maximum(tot, 0.0)
    ys = jnp.dot(h, w_ref[...], preferred_element_type=jnp.float32) * dinv
    o2_ref[0] = ys[:, :dh]
    o2_ref[1] = ys[:, dh:]

  return pl.pallas_call(
      body, out_shape=jax.ShapeDtypeStruct((2, n, dh), jnp.float32)
  )(accp, ys2, dinv, b, w_next)


def _tc_last(accp, ys2, dinv, b):
  _, n, dh = ys2.shape

  def body(acc_ref, ys2_ref, dinv_ref, b_ref, out_ref):
    dinv = dinv_ref[...]
    aggl = acc_ref[0, pl.ds(0, n), :] + ys2_ref[0]
    aggr = acc_ref[1, pl.ds(0, n), :] + ys2_ref[1]
    tot = jnp.concatenate([aggl, aggr], axis=1) * dinv + b_ref[...][None, :]
    out_ref[...] = jnp.maximum(tot, 0.0)

  return pl.pallas_call(
      body, out_shape=jax.ShapeDtypeStruct((n, 2 * dh), jnp.float32)
  )(accp, ys2, dinv, b)


def kernel(x, edge_index, edge_weight, W0, b0, W1, b1):
  n, d = x.shape
  e = edge_index.shape[1]
  kw = e // (NS * EC)
  src2 = edge_index[0].astype(jnp.int32).reshape(NS, kw, EC)
  dst2 = edge_index[1].astype(jnp.int32).reshape(NS, kw, EC)
  ew2 = edge_weight.astype(jnp.float32).reshape(NS, kw, EC)
  n_pad = ((n + NS * LANES - 1) // (NS * LANES)) * NS * LANES
  zeros = jnp.zeros((n_pad // NS, d // 2), jnp.float32)
  epw = e // NW
  dstf = edge_index[1].astype(jnp.int32).reshape(NW, epw)
  ewf = edge_weight.astype(jnp.float32).reshape(NW, epw)

  degp = _sc_degree(dstf, ewf, n_pad)
  ys1, dinv = _tc_first(degp, x, W0)
  accp1 = _sc_aggregate(ys1, src2, dst2, ew2, zeros, n_pad)
  ys2 = _tc_mid(accp1, ys1, dinv, b0, W1)
  accp2 = _sc_aggregate(ys2, src2, dst2, ew2, zeros, n_pad)
  return _tc_last(accp2, ys2, dinv, b1)


# speedup vs baseline: 1.1744x; 1.1744x over previous
import dataclasses
import functools

import jax
import jax.numpy as jnp
from jax import lax
from jax.experimental import pallas as pl
from jax.experimental.pallas import tpu as pltpu
from jax.experimental.pallas import tpu_sc as plsc

NC = 2
NS = 16
NW = NC * NS
LANES = 16

EC = 400


_MESH = plsc.VectorSubcoreMesh(
    core_axis_name="c", subcore_axis_name="s", num_cores=NC, num_subcores=NS
)

_SC_PARAMS = pltpu.CompilerParams(
    needs_layout_passes=False,
    use_tc_tiling_on_sc=False,
    internal_scratch_in_bytes=0,
)


def _bcast_lane(vec, j):
  idx = jnp.full((LANES,), j, dtype=jnp.int32)
  return vec.at[idx].get(mode="promise_in_bounds")


def _bcast_scalar(x):
  return jnp.broadcast_to(x, (LANES,))


def _sc_degree(dstf, ewf, n_pad):
  _, epw = dstf.shape

  @functools.partial(
      pl.kernel,
      out_type=jax.ShapeDtypeStruct((NW, n_pad), jnp.float32),
      mesh=_MESH,
      scratch_types=[
          pltpu.VMEM((epw,), jnp.int32),
          pltpu.VMEM((epw,), jnp.float32),
          pltpu.VMEM((n_pad,), jnp.float32),
          pltpu.SemaphoreType.DMA,
      ],
      compiler_params=_SC_PARAMS,
  )
  def deg_kernel(dst_hbm, ewf_hbm, out_hbm, dst_v, ew_v, deg_v, sem):
    c = lax.axis_index("c")
    s = lax.axis_index("s")
    wid = s * NC + c
    pltpu.sync_copy(dst_hbm.at[wid], dst_v)
    pltpu.sync_copy(ewf_hbm.at[wid], ew_v)

    zero16 = jnp.zeros((LANES,), jnp.float32)

    @pl.loop(0, n_pad // LANES)
    def _(i):
      deg_v[pl.ds(i * LANES, LANES)] = zero16

    @pl.loop(0, epw // LANES)
    def _(i):
      idx = dst_v[pl.ds(i * LANES, LANES)]
      w = ew_v[pl.ds(i * LANES, LANES)]
      plsc.addupdate_scatter(deg_v, [idx], w)

    pltpu.sync_copy(deg_v, out_hbm.at[wid])

  return deg_kernel(dstf, ewf)


@functools.lru_cache(maxsize=None)
def _make_agg_kernel(n, d, kw, c_, n_pad):
  dh = d // 2
  epw = kw * c_
  rpt = n_pad // NS
  assert kw % 2 == 0

  @functools.partial(
      pl.kernel,
      out_type=jax.ShapeDtypeStruct((NC, n_pad, dh), jnp.float32),
      mesh=_MESH,
      scratch_types=[
          pltpu.VMEM((2, c_), jnp.int32),
          pltpu.VMEM((2, c_), jnp.int32),
          pltpu.VMEM((epw,), jnp.float32),
          pltpu.VMEM((c_, dh), jnp.float32),
          pltpu.VMEM((c_, dh), jnp.float32),
          pltpu.VMEM_SHARED((n_pad, dh), jnp.float32),
          pltpu.SemaphoreType.DMA,
          pltpu.SemaphoreType.DMA,
          pltpu.SemaphoreType.DMA,
          pltpu.SemaphoreType.DMA,
      ],
      compiler_params=_SC_PARAMS,
  )
  def agg_kernel(
      ys2_hbm, src_hbm, dst_hbm, ewf_hbm, z_hbm, out_hbm,
      src_v, dst_v, ew_v, rows_a, rows_b, acc_sh,
      gsem_a, gsem_b, ssem_a, ssem_b,
  ):
    c = lax.axis_index("c")
    s = lax.axis_index("s")
    lane_iota = lax.iota(jnp.int32, LANES)
    pltpu.sync_copy(ewf_hbm.at[s], ew_v)

    row0 = s * rpt
    pltpu.sync_copy(z_hbm, acc_sh.at[pl.ds(row0, rpt), :])
    plsc.subcore_barrier()

    ys_c = ys2_hbm.at[c]
    zdrain = z_hbm.at[pl.ds(0, c_), :]

    def scale(buf, i):
      @plsc.parallel_loop(0, c_ // LANES, step=1, unroll=2)
      def _(g):
        ewv = ew_v[pl.ds(i * c_ + g * LANES, LANES)]
        for j in range(LANES):
          bj = _bcast_lane(ewv, j)
          rowv = _bcast_scalar(g * LANES + j)
          for kk in range(dh // LANES):
            col = lane_iota + (kk * LANES)
            val = plsc.load_gather(buf, [rowv, col])
            plsc.store_scatter(buf, [rowv, col], val * bj)

    def drain(buf, sem):
      pltpu.make_async_copy(zdrain, buf, sem).wait()

    pltpu.sync_copy(src_hbm.at[s, 0], src_v.at[0])
    pltpu.async_copy(ys_c.at[src_v.at[0]], rows_a, gsem_a)

    @pl.loop(0, kw // 2)
    def _(t):
      i0 = 2 * t

      @pl.when(t > 0)
      def _():
        drain(rows_b, ssem_b)

      pltpu.sync_copy(src_hbm.at[s, i0 + 1], src_v.at[1])
      pltpu.sync_copy(dst_hbm.at[s, i0], dst_v.at[0])
      pltpu.sync_copy(dst_hbm.at[s, i0 + 1], dst_v.at[1])
      pltpu.async_copy(ys_c.at[src_v.at[1]], rows_b, gsem_b)

      drain(rows_a, gsem_a)
      scale(rows_a, i0)
      pltpu.async_copy(rows_a, acc_sh.at[dst_v.at[0]], ssem_a, add=True)

      drain(rows_b, gsem_b)
      drain(rows_a, ssem_a)

      @pl.when(t < kw // 2 - 1)
      def _():
        pltpu.sync_copy(src_hbm.at[s, i0 + 2], src_v.at[0])
        pltpu.async_copy(ys_c.at[src_v.at[0]], rows_a, gsem_a)

      scale(rows_b, i0 + 1)
      pltpu.async_copy(rows_b, acc_sh.at[dst_v.at[1]], ssem_b, add=True)

    drain(rows_b, ssem_b)

    plsc.subcore_barrier()
    pltpu.sync_copy(
        acc_sh.at[pl.ds(row0, rpt), :], out_hbm.at[c, pl.ds(row0, rpt), :]
    )

  return agg_kernel


def _sc_aggregate(ys2, src2, dst2, ew2, zeros, n_pad):
  _, n, dh = ys2.shape
  _, kw, c_ = src2.shape
  agg_kernel = _make_agg_kernel(n, 2 * dh, kw, c_, n_pad)
  return agg_kernel(ys2, src2, dst2, ew2.reshape(NS, kw * c_), zeros)


def _tc_first(degp, x, w0):
  n, d = x.shape
  dh = d // 2

  def body(degp_ref, x_ref, w_ref, ys2_ref, dinv_ref):
    deg = jnp.sum(degp_ref[:, pl.ds(0, n)], axis=0)[:, None] + 1.0
    dinv = lax.rsqrt(deg)
    dinv_ref[...] = dinv
    xw = jnp.dot(x_ref[...], w_ref[...], preferred_element_type=jnp.float32)
    ys = xw * dinv
    ys2_ref[0] = ys[:, :dh]
    ys2_ref[1] = ys[:, dh:]

  return pl.pallas_call(
      body,
      out_shape=(
          jax.ShapeDtypeStruct((2, n, dh), jnp.float32),
          jax.ShapeDtypeStruct((n, 1), jnp.float32),
      ),
  )(degp, x, w0)


def _tc_mid(accp, ys2, dinv, b, w_next):
  _, n, dh = ys2.shape

  def body(acc_ref, ys2_ref, dinv_ref, b_ref, w_ref, o2_ref):
    dinv = dinv_ref[...]
    aggl = acc_ref[0, pl.ds(0, n), :] + ys2_ref[0]
    aggr = acc_ref[1, pl.ds(0, n), :] + ys2_ref[1]
    tot = jnp.concatenate([aggl, aggr], axis=1) * dinv + b_ref[...][None, :]
    h = jnp.maximum(tot, 0.0)
    ys = jnp.dot(h, w_ref[...], preferred_element_type=jnp.float32) * dinv
    o2_ref[0] = ys[:, :dh]
    o2_ref[1] = ys[:, dh:]

  return pl.pallas_call(
      body, out_shape=jax.ShapeDtypeStruct((2, n, dh), jnp.float32)
  )(accp, ys2, dinv, b, w_next)


def _tc_last(accp, ys2, dinv, b):
  _, n, dh = ys2.shape

  def body(acc_ref, ys2_ref, dinv_ref, b_ref, out_ref):
    dinv = dinv_ref[...]
    aggl = acc_ref[0, pl.ds(0, n), :] + ys2_ref[0]
    aggr = acc_ref[1, pl.ds(0, n), :] + ys2_ref[1]
    tot = jnp.concatenate([aggl, aggr], axis=1) * dinv + b_ref[...][None, :]
    out_ref[...] = jnp.maximum(tot, 0.0)

  return pl.pallas_call(
      body, out_shape=jax.ShapeDtypeStruct((n, 2 * dh), jnp.float32)
  )(accp, ys2, dinv, b)


def kernel(x, edge_index, edge_weight, W0, b0, W1, b1):
  n, d = x.shape
  e = edge_index.shape[1]
  kw = e // (NS * EC)
  src2 = edge_index[0].astype(jnp.int32).reshape(NS, kw, EC)
  dst2 = edge_index[1].astype(jnp.int32).reshape(NS, kw, EC)
  ew2 = edge_weight.astype(jnp.float32).reshape(NS, kw, EC)
  n_pad = ((n + NS * LANES - 1) // (NS * LANES)) * NS * LANES
  zeros = jnp.zeros((n_pad // NS, d // 2), jnp.float32)
  epw = e // NW
  dstf = edge_index[1].astype(jnp.int32).reshape(NW, epw)
  ewf = edge_weight.astype(jnp.float32).reshape(NW, epw)

  degp = _sc_degree(dstf, ewf, n_pad)
  ys1, dinv = _tc_first(degp, x, W0)
  accp1 = _sc_aggregate(ys1, src2, dst2, ew2, zeros, n_pad)
  ys2 = _tc_mid(accp1, ys1, dinv, b0, W1)
  accp2 = _sc_aggregate(ys2, src2, dst2, ew2, zeros, n_pad)
  return _tc_last(accp2, ys2, dinv, b1)


# speedup vs baseline: 1.2717x; 1.0828x over previous
import dataclasses
import functools

import jax
import jax.numpy as jnp
from jax import lax
from jax.experimental import pallas as pl
from jax.experimental.pallas import tpu as pltpu
from jax.experimental.pallas import tpu_sc as plsc

NC = 2
NS = 16
NW = NC * NS
LANES = 16

EC = 400


_MESH = plsc.VectorSubcoreMesh(
    core_axis_name="c", subcore_axis_name="s", num_cores=NC, num_subcores=NS
)

_SC_PARAMS = pltpu.CompilerParams(
    needs_layout_passes=False,
    use_tc_tiling_on_sc=False,
    internal_scratch_in_bytes=0,
)


def _bcast_lane(vec, j):
  idx = jnp.full((LANES,), j, dtype=jnp.int32)
  return vec.at[idx].get(mode="promise_in_bounds")


def _bcast_scalar(x):
  return jnp.broadcast_to(x, (LANES,))


def _sc_degree(dstf, ewf, n_pad):
  _, epw = dstf.shape

  @functools.partial(
      pl.kernel,
      out_type=jax.ShapeDtypeStruct((NW, n_pad), jnp.float32),
      mesh=_MESH,
      scratch_types=[
          pltpu.VMEM((epw,), jnp.int32),
          pltpu.VMEM((epw,), jnp.float32),
          pltpu.VMEM((n_pad,), jnp.float32),
          pltpu.SemaphoreType.DMA,
      ],
      compiler_params=_SC_PARAMS,
  )
  def deg_kernel(dst_hbm, ewf_hbm, out_hbm, dst_v, ew_v, deg_v, sem):
    c = lax.axis_index("c")
    s = lax.axis_index("s")
    wid = s * NC + c
    pltpu.sync_copy(dst_hbm.at[wid], dst_v)
    pltpu.sync_copy(ewf_hbm.at[wid], ew_v)

    zero16 = jnp.zeros((LANES,), jnp.float32)

    @pl.loop(0, n_pad // LANES)
    def _(i):
      deg_v[pl.ds(i * LANES, LANES)] = zero16

    @pl.loop(0, epw // LANES)
    def _(i):
      idx = dst_v[pl.ds(i * LANES, LANES)]
      w = ew_v[pl.ds(i * LANES, LANES)]
      plsc.addupdate_scatter(deg_v, [idx], w)

    pltpu.sync_copy(deg_v, out_hbm.at[wid])

  return deg_kernel(dstf, ewf)


@functools.lru_cache(maxsize=None)
def _make_agg_kernel(n, d, kw, c_, n_pad):
  dh = d // 2
  epw = kw * c_
  rpt = n_pad // NS
  assert kw % 2 == 0

  @functools.partial(
      pl.kernel,
      out_type=jax.ShapeDtypeStruct((NC, n_pad, dh), jnp.float32),
      mesh=_MESH,
      scratch_types=[
          pltpu.VMEM((kw, c_), jnp.int32),
          pltpu.VMEM((kw, c_), jnp.int32),
          pltpu.VMEM((epw,), jnp.float32),
          pltpu.VMEM((c_, dh), jnp.float32),
          pltpu.VMEM_SHARED((n_pad, dh), jnp.float32),
          pltpu.SemaphoreType.DMA,
      ],
      compiler_params=_SC_PARAMS,
  )
  def agg_kernel(
      ys2_hbm, src_hbm, dst_hbm, ewf_hbm, z_hbm, out_hbm,
      src_v, dst_v, ew_v, rows_v, acc_sh, gsem,
  ):
    c = lax.axis_index("c")
    s = lax.axis_index("s")
    lane_iota = lax.iota(jnp.int32, LANES)
    pltpu.sync_copy(src_hbm.at[s], src_v)
    pltpu.sync_copy(dst_hbm.at[s], dst_v)
    pltpu.sync_copy(ewf_hbm.at[s], ew_v)

    row0 = s * rpt
    pltpu.sync_copy(z_hbm, acc_sh.at[pl.ds(row0, rpt), :])
    plsc.subcore_barrier()

    ys_c = ys2_hbm.at[c]

    def scale(buf, i):
      @plsc.parallel_loop(0, c_ // LANES, step=1, unroll=2)
      def _(g):
        ewv = ew_v[pl.ds(i * c_ + g * LANES, LANES)]
        for j in range(LANES):
          bj = _bcast_lane(ewv, j)
          rowv = _bcast_scalar(g * LANES + j)
          for kk in range(dh // LANES):
            col = lane_iota + (kk * LANES)
            val = plsc.load_gather(buf, [rowv, col])
            plsc.store_scatter(buf, [rowv, col], val * bj)

    @pl.loop(0, kw)
    def _(i):
      pltpu.async_copy(ys_c.at[src_v.at[i]], rows_v, gsem).wait()
      scale(rows_v, i)
      pltpu.sync_copy(rows_v, acc_sh.at[dst_v.at[i]], add=True)

    plsc.subcore_barrier()
    pltpu.sync_copy(
        acc_sh.at[pl.ds(row0, rpt), :], out_hbm.at[c, pl.ds(row0, rpt), :]
    )

  return agg_kernel


def _sc_aggregate(ys2, src2, dst2, ew2, zeros, n_pad):
  _, n, dh = ys2.shape
  _, kw, c_ = src2.shape
  agg_kernel = _make_agg_kernel(n, 2 * dh, kw, c_, n_pad)
  return agg_kernel(ys2, src2, dst2, ew2.reshape(NS, kw * c_), zeros)


def _tc_first(degp, x, w0):
  n, d = x.shape
  dh = d // 2

  def body(degp_ref, x_ref, w_ref, ys2_ref, dinv_ref):
    deg = jnp.sum(degp_ref[:, pl.ds(0, n)], axis=0)[:, None] + 1.0
    dinv = lax.rsqrt(deg)
    dinv_ref[...] = dinv
    xw = jnp.dot(x_ref[...], w_ref[...], preferred_element_type=jnp.float32)
    ys = xw * dinv
    ys2_ref[0] = ys[:, :dh]
    ys2_ref[1] = ys[:, dh:]

  return pl.pallas_call(
      body,
      out_shape=(
          jax.ShapeDtypeStruct((2, n, dh), jnp.float32),
          jax.ShapeDtypeStruct((n, 1), jnp.float32),
      ),
  )(degp, x, w0)


def _tc_mid(accp, ys2, dinv, b, w_next):
  _, n, dh = ys2.shape

  def body(acc_ref, ys2_ref, dinv_ref, b_ref, w_ref, o2_ref):
    dinv = dinv_ref[...]
    aggl = acc_ref[0, pl.ds(0, n), :] + ys2_ref[0]
    aggr = acc_ref[1, pl.ds(0, n), :] + ys2_ref[1]
    tot = jnp.concatenate([aggl, aggr], axis=1) * dinv + b_ref[...][None, :]
    h = jnp.maximum(tot, 0.0)
    ys = jnp.dot(h, w_ref[...], preferred_element_type=jnp.float32) * dinv
    o2_ref[0] = ys[:, :dh]
    o2_ref[1] = ys[:, dh:]

  return pl.pallas_call(
      body, out_shape=jax.ShapeDtypeStruct((2, n, dh), jnp.float32)
  )(accp, ys2, dinv, b, w_next)


def _tc_last(accp, ys2, dinv, b):
  _, n, dh = ys2.shape

  def body(acc_ref, ys2_ref, dinv_ref, b_ref, out_ref):
    dinv = dinv_ref[...]
    aggl = acc_ref[0, pl.ds(0, n), :] + ys2_ref[0]
    aggr = acc_ref[1, pl.ds(0, n), :] + ys2_ref[1]
    tot = jnp.concatenate([aggl, aggr], axis=1) * dinv + b_ref[...][None, :]
    out_ref[...] = jnp.maximum(tot, 0.0)

  return pl.pallas_call(
      body, out_shape=jax.ShapeDtypeStruct((n, 2 * dh), jnp.float32)
  )(accp, ys2, dinv, b)


def kernel(x, edge_index, edge_weight, W0, b0, W1, b1):
  n, d = x.shape
  e = edge_index.shape[1]
  kw = e // (NS * EC)
  src2 = edge_index[0].astype(jnp.int32).reshape(NS, kw, EC)
  dst2 = edge_index[1].astype(jnp.int32).reshape(NS, kw, EC)
  ew2 = edge_weight.astype(jnp.float32).reshape(NS, kw, EC)
  n_pad = ((n + NS * LANES - 1) // (NS * LANES)) * NS * LANES
  zeros = jnp.zeros((n_pad // NS, d // 2), jnp.float32)
  epw = e // NW
  dstf = edge_index[1].astype(jnp.int32).reshape(NW, epw)
  ewf = edge_weight.astype(jnp.float32).reshape(NW, epw)

  degp = _sc_degree(dstf, ewf, n_pad)
  ys1, dinv = _tc_first(degp, x, W0)
  accp1 = _sc_aggregate(ys1, src2, dst2, ew2, zeros, n_pad)
  ys2 = _tc_mid(accp1, ys1, dinv, b0, W1)
  accp2 = _sc_aggregate(ys2, src2, dst2, ew2, zeros, n_pad)
  return _tc_last(accp2, ys2, dinv, b1)
